# baseline (speedup 1.0000x reference)
import functools

import jax
import jax.numpy as jnp
from jax import lax
from jax.experimental import pallas as pl
from jax.experimental.pallas import tpu as pltpu
from jax.experimental.pallas import tpu_sc as plsc

_D = 64
_B = 16384
_T = 2 * _B
_NW = 32
_ROWS_PER_W = _T // _NW
_CHUNK = 128
_NCHUNK = _ROWS_PER_W // _CHUNK
_MARGIN = 1.0
_C_REG = 0.1


def _pre_body(rel_ref, nv_ref, q_ref):
    nv = nv_ref[...]
    nn = jnp.sum(nv * nv, axis=1, keepdims=True)
    nhat = nv / jnp.maximum(jnp.sqrt(nn), 1e-12)
    q_ref[...] = jnp.concatenate([rel_ref[...], nhat], axis=1)


def _make_q(rel, nv):
    return pl.pallas_call(
        _pre_body,
        out_shape=jax.ShapeDtypeStruct((rel.shape[0], 2 * _D), jnp.float32),
    )(rel, nv)


_HALF = 500224
_TCOLS = 512
_TGRID = _HALF // _TCOLS


def _pack_body(in1_ref, in2_ref, out_ref):
    out_ref[...] = jnp.concatenate([in1_ref[...].T, in2_ref[...].T], axis=1)


def _pack_pairs(ent_t):
    return pl.pallas_call(
        _pack_body,
        grid=(_TGRID,),
        in_specs=[
            pl.BlockSpec((_D, _TCOLS), lambda i: (0, i)),
            pl.BlockSpec((_D, _TCOLS), lambda i: (0, i + _TGRID)),
        ],
        out_specs=pl.BlockSpec((_TCOLS, 2 * _D), lambda i: (i, 0)),
        out_shape=jax.ShapeDtypeStruct((_HALF, 2 * _D), jnp.float32),
    )(ent_t, ent_t)


def _sc_body(h2_hbm, t2_hbm, r_hbm, h_hbm, t_hbm, ent2_hbm, q_hbm, out_hbm,
             h2idx_v, t2idx_v, ridx_v, hidx_v, tidx_v,
             h_a, t_a, q_a, h_b, t_b, q_b, oacc,
             sem_a, sem_b, semq_a, semq_b):
    wid = lax.axis_index("s") * 2 + lax.axis_index("c")
    base = wid * _ROWS_PER_W
    lane = lax.iota(jnp.int32, 16)
    zero = jnp.zeros((16,), jnp.float32)
    one = jnp.full((16,), 1, jnp.int32)

    pltpu.sync_copy(h2_hbm.at[pl.ds(base, _ROWS_PER_W)], h2idx_v)
    pltpu.sync_copy(t2_hbm.at[pl.ds(base, _ROWS_PER_W)], t2idx_v)
    pltpu.sync_copy(r_hbm.at[pl.ds(base, _ROWS_PER_W)], ridx_v)
    pltpu.sync_copy(h_hbm.at[pl.ds(base, _ROWS_PER_W)], hidx_v)
    pltpu.sync_copy(t_hbm.at[pl.ds(base, _ROWS_PER_W)], tidx_v)

    def fire(c, hX, tX, qX, semX, semqX):
        pltpu.async_copy(
            ent2_hbm.at[h2idx_v.at[pl.ds(c * _CHUNK, _CHUNK)]], hX, semX)
        pltpu.async_copy(
            ent2_hbm.at[t2idx_v.at[pl.ds(c * _CHUNK, _CHUNK)]], tX, semX)
        pltpu.async_copy(
            q_hbm.at[ridx_v.at[pl.ds(c * _CHUNK, _CHUNK)]], qX, semqX)

    def drain(hX, tX, qX, semX, semqX):
        pltpu.make_async_copy(ent2_hbm.at[pl.ds(0, _CHUNK)], hX, semX).wait()
        pltpu.make_async_copy(ent2_hbm.at[pl.ds(0, _CHUNK)], tX, semX).wait()
        pltpu.make_async_copy(q_hbm.at[pl.ds(0, _CHUNK)], qX, semqX).wait()

    def compute(c, hX, tX, qX):
        def group_body(g, gcarry):
            rows = jnp.full((16,), g * 16, jnp.int32) + lane
            halfv = jnp.full((16,), _HALF, jnp.int32)
            dv = jnp.full((16,), _D, jnp.int32)
            zv = jnp.zeros((16,), jnp.int32)
            hoff = jnp.where(
                hidx_v[pl.ds(c * _CHUNK + g * 16, 16)] < halfv, zv, dv)
            toff = jnp.where(
                tidx_v[pl.ds(c * _CHUNK + g * 16, 16)] < halfv, zv, dv)
            jv = jnp.zeros((16,), jnp.int32)
            cacc = zero
            for _ in range(_D):
                gh = plsc.load_gather(hX, [rows, hoff + jv])
                gt = plsc.load_gather(tX, [rows, toff + jv])
                gn = plsc.load_gather(qX, [rows, jv + _D])
                cacc = cacc + (gh - gt) * gn
                jv = jv + one
            jv = jnp.zeros((16,), jnp.int32)
            ssacc = zero
            for _ in range(_D):
                gh = plsc.load_gather(hX, [rows, hoff + jv])
                gt = plsc.load_gather(tX, [rows, toff + jv])
                gn = plsc.load_gather(qX, [rows, jv + _D])
                gr = plsc.load_gather(qX, [rows, jv])
                d = (gh - gt) + gr - cacc * gn
                ssacc = ssacc + d * d
                jv = jv + one
            oacc[pl.ds(c * _CHUNK + g * 16, 16)] = ssacc
            return gcarry

        lax.fori_loop(0, _CHUNK // 16, group_body, 0)

    fire(0, h_a, t_a, q_a, sem_a, semq_a)

    def pair_body(p, carry):
        c0 = 2 * p
        fire(c0 + 1, h_b, t_b, q_b, sem_b, semq_b)
        drain(h_a, t_a, q_a, sem_a, semq_a)
        compute(c0, h_a, t_a, q_a)

        @pl.when(p < _NCHUNK // 2 - 1)
        def _():
            fire(c0 + 2, h_a, t_a, q_a, sem_a, semq_a)

        drain(h_b, t_b, q_b, sem_b, semq_b)
        compute(c0 + 1, h_b, t_b, q_b)
        return carry

    lax.fori_loop(0, _NCHUNK // 2, pair_body, 0)
    pltpu.sync_copy(oacc, out_hbm.at[pl.ds(base, _ROWS_PER_W)])


def _sc_scores(h2, t2, r_idx, h_idx, t_idx, ent2, q):
    mesh = plsc.VectorSubcoreMesh(core_axis_name="c", subcore_axis_name="s")
    fn = functools.partial(
        pl.kernel,
        out_type=jax.ShapeDtypeStruct((_T,), jnp.float32),
        mesh=mesh,
        scratch_types=[
            pltpu.VMEM((_ROWS_PER_W,), jnp.int32),
            pltpu.VMEM((_ROWS_PER_W,), jnp.int32),
            pltpu.VMEM((_ROWS_PER_W,), jnp.int32),
            pltpu.VMEM((_ROWS_PER_W,), jnp.int32),
            pltpu.VMEM((_ROWS_PER_W,), jnp.int32),
            pltpu.VMEM((_CHUNK, 2 * _D), jnp.float32),
            pltpu.VMEM((_CHUNK, 2 * _D), jnp.float32),
            pltpu.VMEM((_CHUNK, 2 * _D), jnp.float32),
            pltpu.VMEM((_CHUNK, 2 * _D), jnp.float32),
            pltpu.VMEM((_CHUNK, 2 * _D), jnp.float32),
            pltpu.VMEM((_CHUNK, 2 * _D), jnp.float32),
            pltpu.VMEM((_ROWS_PER_W,), jnp.float32),
            pltpu.SemaphoreType.DMA,
            pltpu.SemaphoreType.DMA,
            pltpu.SemaphoreType.DMA,
            pltpu.SemaphoreType.DMA,
        ],
        compiler_params=pltpu.CompilerParams(
            needs_layout_passes=False, use_tc_tiling_on_sc=True),
    )(_sc_body)
    return fn(h2, t2, r_idx, h_idx, t_idx, ent2, q)


def _post_body(ss_ref, rel_ref, nv_ref, out_ref):
    s = jnp.sqrt(ss_ref[...])
    basic = jnp.mean(jnp.maximum(_MARGIN + s[:128, :] - s[128:, :], 0.0))
    rel = rel_ref[...]
    nv = nv_ref[...]
    rn = jnp.sqrt(jnp.sum(rel * rel, axis=1))
    wn = jnp.sqrt(jnp.sum(nv * nv, axis=1))
    cons = jnp.sum(jnp.abs(jnp.sum(rel * nv, axis=1) / (rn * wn)))
    out_ref[...] = jnp.broadcast_to(basic + _C_REG * cons, (1, 1))


def _post(ss, rel, nv):
    out = pl.pallas_call(
        _post_body,
        out_shape=jax.ShapeDtypeStruct((1, 1), jnp.float32),
    )(ss.reshape(_T // 128, 128), rel, nv)
    return out[0, 0]


def kernel(pos_h, pos_r, pos_t, neg_h, neg_r, neg_t,
           entity_embedding, relation_embedding, normal_vector):
    h_idx = jnp.concatenate([pos_h, neg_h]).astype(jnp.int32)
    t_idx = jnp.concatenate([pos_t, neg_t]).astype(jnp.int32)
    r_idx = jnp.concatenate([pos_r, neg_r]).astype(jnp.int32)
    ent2 = _pack_pairs(entity_embedding.T)
    q = _make_q(relation_embedding, normal_vector)
    h2 = jnp.where(h_idx < _HALF, h_idx, h_idx - _HALF)
    t2 = jnp.where(t_idx < _HALF, t_idx, t_idx - _HALF)
    ss = _sc_scores(h2, t2, r_idx, h_idx, t_idx, ent2, q)
    return _post(ss, relation_embedding, normal_vector)


# speedup vs baseline: 1.6573x; 1.6573x over previous
import functools

import jax
import jax.numpy as jnp
from jax import lax
from jax.experimental import pallas as pl
from jax.experimental.pallas import tpu as pltpu
from jax.experimental.pallas import tpu_sc as plsc

_D = 64
_B = 16384
_T = 2 * _B
_NW = 32
_ROWS_PER_W = _T // _NW
_CHUNK = 128
_NCHUNK = _ROWS_PER_W // _CHUNK
_MARGIN = 1.0
_C_REG = 0.1


def _pre_body(rel_ref, nv_ref, q_ref):
    nv = nv_ref[...]
    nn = jnp.sum(nv * nv, axis=1, keepdims=True)
    nhat = nv / jnp.maximum(jnp.sqrt(nn), 1e-12)
    q_ref[...] = jnp.concatenate([rel_ref[...], nhat], axis=1)


def _make_q(rel, nv):
    return pl.pallas_call(
        _pre_body,
        out_shape=jax.ShapeDtypeStruct((rel.shape[0], 2 * _D), jnp.float32),
    )(rel, nv)


def _sc_body(h_hbm, t_hbm, r_hbm, ent_hbm, q_hbm, out_hbm,
             hidx_v, tidx_v, ridx_v,
             h_a, t_a, q_a, h_b, t_b, q_b, oacc,
             sem_a, sem_b, semq_a, semq_b):
    wid = lax.axis_index("s") * 2 + lax.axis_index("c")
    base = wid * _ROWS_PER_W
    lane = lax.iota(jnp.int32, 16)
    zero = jnp.zeros((16,), jnp.float32)
    one = jnp.full((16,), 1, jnp.int32)

    pltpu.sync_copy(h_hbm.at[pl.ds(base, _ROWS_PER_W)], hidx_v)
    pltpu.sync_copy(t_hbm.at[pl.ds(base, _ROWS_PER_W)], tidx_v)
    pltpu.sync_copy(r_hbm.at[pl.ds(base, _ROWS_PER_W)], ridx_v)

    def fire(c, hX, tX, qX, semX, semqX):
        pltpu.async_copy(
            q_hbm.at[ridx_v.at[pl.ds(c * _CHUNK, _CHUNK)]], qX, semqX)

        def fire_v(v, carry):
            hvec = hidx_v[pl.ds(c * _CHUNK + v * 16, 16)]
            tvec = tidx_v[pl.ds(c * _CHUNK + v * 16, 16)]
            for l in range(16):
                pltpu.async_copy(
                    ent_hbm.at[pl.ds(hvec[l], 1)],
                    hX.at[pl.ds(v * 16 + l, 1)], semX)
                pltpu.async_copy(
                    ent_hbm.at[pl.ds(tvec[l], 1)],
                    tX.at[pl.ds(v * 16 + l, 1)], semX)
            return carry

        lax.fori_loop(0, _CHUNK // 16, fire_v, 0)

    def drain(hX, tX, qX, semX, semqX):
        pltpu.make_async_copy(ent_hbm.at[pl.ds(0, _CHUNK)], hX, semX).wait()
        pltpu.make_async_copy(ent_hbm.at[pl.ds(0, _CHUNK)], tX, semX).wait()
        pltpu.make_async_copy(q_hbm.at[pl.ds(0, _CHUNK)], qX, semqX).wait()

    def compute(c, hX, tX, qX):
        def group_body(g, gcarry):
            rows = jnp.full((16,), g * 16, jnp.int32) + lane
            jv = jnp.zeros((16,), jnp.int32)
            cacc = zero
            for _ in range(_D):
                gh = plsc.load_gather(hX, [rows, jv])
                gt = plsc.load_gather(tX, [rows, jv])
                gn = plsc.load_gather(qX, [rows, jv + _D])
                cacc = cacc + (gh - gt) * gn
                jv = jv + one
            jv = jnp.zeros((16,), jnp.int32)
            ssacc = zero
            for _ in range(_D):
                gh = plsc.load_gather(hX, [rows, jv])
                gt = plsc.load_gather(tX, [rows, jv])
                gn = plsc.load_gather(qX, [rows, jv + _D])
                gr = plsc.load_gather(qX, [rows, jv])
                d = (gh - gt) + gr - cacc * gn
                ssacc = ssacc + d * d
                jv = jv + one
            oacc[pl.ds(c * _CHUNK + g * 16, 16)] = ssacc
            return gcarry

        lax.fori_loop(0, _CHUNK // 16, group_body, 0)

    fire(0, h_a, t_a, q_a, sem_a, semq_a)

    def pair_body(p, carry):
        c0 = 2 * p
        fire(c0 + 1, h_b, t_b, q_b, sem_b, semq_b)
        drain(h_a, t_a, q_a, sem_a, semq_a)
        compute(c0, h_a, t_a, q_a)

        @pl.when(p < _NCHUNK // 2 - 1)
        def _():
            fire(c0 + 2, h_a, t_a, q_a, sem_a, semq_a)

        drain(h_b, t_b, q_b, sem_b, semq_b)
        compute(c0 + 1, h_b, t_b, q_b)
        return carry

    lax.fori_loop(0, _NCHUNK // 2, pair_body, 0)
    pltpu.sync_copy(oacc, out_hbm.at[pl.ds(base, _ROWS_PER_W)])


def _sc_scores(h_idx, t_idx, r_idx, ent, q):
    mesh = plsc.VectorSubcoreMesh(core_axis_name="c", subcore_axis_name="s")
    fn = functools.partial(
        pl.kernel,
        out_type=jax.ShapeDtypeStruct((_T,), jnp.float32),
        mesh=mesh,
        scratch_types=[
            pltpu.VMEM((_ROWS_PER_W,), jnp.int32),
            pltpu.VMEM((_ROWS_PER_W,), jnp.int32),
            pltpu.VMEM((_ROWS_PER_W,), jnp.int32),
            pltpu.VMEM((_CHUNK, _D), jnp.float32),
            pltpu.VMEM((_CHUNK, _D), jnp.float32),
            pltpu.VMEM((_CHUNK, 2 * _D), jnp.float32),
            pltpu.VMEM((_CHUNK, _D), jnp.float32),
            pltpu.VMEM((_CHUNK, _D), jnp.float32),
            pltpu.VMEM((_CHUNK, 2 * _D), jnp.float32),
            pltpu.VMEM((_ROWS_PER_W,), jnp.float32),
            pltpu.SemaphoreType.DMA,
            pltpu.SemaphoreType.DMA,
            pltpu.SemaphoreType.DMA,
            pltpu.SemaphoreType.DMA,
        ],
        compiler_params=pltpu.CompilerParams(
            needs_layout_passes=False, use_tc_tiling_on_sc=True,
            disable_bounds_checks=True, disable_semaphore_checks=True),
    )(_sc_body)
    return fn(h_idx, t_idx, r_idx, ent, q)


def _post_body(ss_ref, rel_ref, nv_ref, out_ref):
    s = jnp.sqrt(ss_ref[...])
    basic = jnp.mean(jnp.maximum(_MARGIN + s[:128, :] - s[128:, :], 0.0))
    rel = rel_ref[...]
    nv = nv_ref[...]
    rn = jnp.sqrt(jnp.sum(rel * rel, axis=1))
    wn = jnp.sqrt(jnp.sum(nv * nv, axis=1))
    cons = jnp.sum(jnp.abs(jnp.sum(rel * nv, axis=1) / (rn * wn)))
    out_ref[...] = jnp.broadcast_to(basic + _C_REG * cons, (1, 1))


def _post(ss, rel, nv):
    out = pl.pallas_call(
        _post_body,
        out_shape=jax.ShapeDtypeStruct((1, 1), jnp.float32),
    )(ss.reshape(_T // 128, 128), rel, nv)
    return out[0, 0]


def kernel(pos_h, pos_r, pos_t, neg_h, neg_r, neg_t,
           entity_embedding, relation_embedding, normal_vector):
    h_idx = jnp.concatenate([pos_h, neg_h]).astype(jnp.int32)
    t_idx = jnp.concatenate([pos_t, neg_t]).astype(jnp.int32)
    r_idx = jnp.concatenate([pos_r, neg_r]).astype(jnp.int32)
    q = _make_q(relation_embedding, normal_vector)
    ss = _sc_scores(h_idx, t_idx, r_idx, entity_embedding, q)
    return _post(ss, relation_embedding, normal_vector)


# speedup vs baseline: 1.6650x; 1.0047x over previous
import functools

import jax
import jax.numpy as jnp
from jax import lax
from jax.experimental import pallas as pl
from jax.experimental.pallas import tpu as pltpu
from jax.experimental.pallas import tpu_sc as plsc

_D = 64
_B = 16384
_T = 2 * _B
_NW = 32
_ROWS_PER_W = _T // _NW
_CHUNK = 128
_NCHUNK = _ROWS_PER_W // _CHUNK
_MARGIN = 1.0
_C_REG = 0.1


def _pre_body(rel_ref, nv_ref, q_ref):
    nv = nv_ref[...]
    nn = jnp.sum(nv * nv, axis=1, keepdims=True)
    nhat = nv / jnp.maximum(jnp.sqrt(nn), 1e-12)
    q_ref[...] = jnp.concatenate([rel_ref[...], nhat], axis=1)


def _make_q(rel, nv):
    return pl.pallas_call(
        _pre_body,
        out_shape=jax.ShapeDtypeStruct((rel.shape[0], 2 * _D), jnp.float32),
    )(rel, nv)


def _sc_body(ph_hbm, pt_hbm, pr_hbm, nh_hbm, nt_hbm, nr_hbm,
             ent_hbm, q_hbm, out_hbm,
             hidx_v, tidx_v, ridx_v,
             h_a, t_a, q_a, h_b, t_b, q_b, oacc,
             sem_a, sem_b, semq_a, semq_b):
    wid = lax.axis_index("s") * 2 + lax.axis_index("c")
    lane = lax.iota(jnp.int32, 16)
    zero = jnp.zeros((16,), jnp.float32)
    one = jnp.full((16,), 1, jnp.int32)

    @pl.when(wid < _NW // 2)
    def _():
        pbase = wid * _ROWS_PER_W
        pltpu.sync_copy(ph_hbm.at[pl.ds(pbase, _ROWS_PER_W)], hidx_v)
        pltpu.sync_copy(pt_hbm.at[pl.ds(pbase, _ROWS_PER_W)], tidx_v)
        pltpu.sync_copy(pr_hbm.at[pl.ds(pbase, _ROWS_PER_W)], ridx_v)

    @pl.when(wid >= _NW // 2)
    def _():
        nbase = (wid - _NW // 2) * _ROWS_PER_W
        pltpu.sync_copy(nh_hbm.at[pl.ds(nbase, _ROWS_PER_W)], hidx_v)
        pltpu.sync_copy(nt_hbm.at[pl.ds(nbase, _ROWS_PER_W)], tidx_v)
        pltpu.sync_copy(nr_hbm.at[pl.ds(nbase, _ROWS_PER_W)], ridx_v)

    def fire(c, hX, tX, qX, semX, semqX):
        pltpu.async_copy(
            q_hbm.at[ridx_v.at[pl.ds(c * _CHUNK, _CHUNK)]], qX, semqX)

        def fire_v(v, carry):
            hvec = hidx_v[pl.ds(c * _CHUNK + v * 16, 16)]
            tvec = tidx_v[pl.ds(c * _CHUNK + v * 16, 16)]
            for l in range(16):
                pltpu.async_copy(
                    ent_hbm.at[pl.ds(hvec[l], 1)],
                    hX.at[pl.ds(v * 16 + l, 1)], semX)
                pltpu.async_copy(
                    ent_hbm.at[pl.ds(tvec[l], 1)],
                    tX.at[pl.ds(v * 16 + l, 1)], semX)
            return carry

        lax.fori_loop(0, _CHUNK // 16, fire_v, 0)

    def drain(hX, tX, qX, semX, semqX):
        pltpu.make_async_copy(ent_hbm.at[pl.ds(0, _CHUNK)], hX, semX).wait()
        pltpu.make_async_copy(ent_hbm.at[pl.ds(0, _CHUNK)], tX, semX).wait()
        pltpu.make_async_copy(q_hbm.at[pl.ds(0, _CHUNK)], qX, semqX).wait()

    def compute(c, hX, tX, qX):
        def group_body(g, gcarry):
            rows = jnp.full((16,), g * 16, jnp.int32) + lane
            jv = jnp.zeros((16,), jnp.int32)
            cacc = zero
            for _ in range(_D):
                gh = plsc.load_gather(hX, [rows, jv])
                gt = plsc.load_gather(tX, [rows, jv])
                gn = plsc.load_gather(qX, [rows, jv + _D])
                cacc = cacc + (gh - gt) * gn
                jv = jv + one
            jv = jnp.zeros((16,), jnp.int32)
            ssacc = zero
            for _ in range(_D):
                gh = plsc.load_gather(hX, [rows, jv])
                gt = plsc.load_gather(tX, [rows, jv])
                gn = plsc.load_gather(qX, [rows, jv + _D])
                gr = plsc.load_gather(qX, [rows, jv])
                d = (gh - gt) + gr - cacc * gn
                ssacc = ssacc + d * d
                jv = jv + one
            oacc[c, pl.ds(g * 16, 16)] = ssacc
            return gcarry

        lax.fori_loop(0, _CHUNK // 16, group_body, 0)

    fire(0, h_a, t_a, q_a, sem_a, semq_a)

    def pair_body(p, carry):
        c0 = 2 * p
        fire(c0 + 1, h_b, t_b, q_b, sem_b, semq_b)
        drain(h_a, t_a, q_a, sem_a, semq_a)
        compute(c0, h_a, t_a, q_a)

        @pl.when(p < _NCHUNK // 2 - 1)
        def _():
            fire(c0 + 2, h_a, t_a, q_a, sem_a, semq_a)

        drain(h_b, t_b, q_b, sem_b, semq_b)
        compute(c0 + 1, h_b, t_b, q_b)
        return carry

    lax.fori_loop(0, _NCHUNK // 2, pair_body, 0)
    pltpu.sync_copy(oacc, out_hbm.at[pl.ds(wid * _NCHUNK, _NCHUNK), :])


def _sc_scores(ph, pt, pr, nh, nt, nr, ent, q):
    mesh = plsc.VectorSubcoreMesh(core_axis_name="c", subcore_axis_name="s")
    fn = functools.partial(
        pl.kernel,
        out_type=jax.ShapeDtypeStruct((_T // _CHUNK, _CHUNK), jnp.float32),
        mesh=mesh,
        scratch_types=[
            pltpu.VMEM((_ROWS_PER_W,), jnp.int32),
            pltpu.VMEM((_ROWS_PER_W,), jnp.int32),
            pltpu.VMEM((_ROWS_PER_W,), jnp.int32),
            pltpu.VMEM((_CHUNK, _D), jnp.float32),
            pltpu.VMEM((_CHUNK, _D), jnp.float32),
            pltpu.VMEM((_CHUNK, 2 * _D), jnp.float32),
            pltpu.VMEM((_CHUNK, _D), jnp.float32),
            pltpu.VMEM((_CHUNK, _D), jnp.float32),
            pltpu.VMEM((_CHUNK, 2 * _D), jnp.float32),
            pltpu.VMEM((_NCHUNK, _CHUNK), jnp.float32),
            pltpu.SemaphoreType.DMA,
            pltpu.SemaphoreType.DMA,
            pltpu.SemaphoreType.DMA,
            pltpu.SemaphoreType.DMA,
        ],
        compiler_params=pltpu.CompilerParams(
            needs_layout_passes=False, use_tc_tiling_on_sc=True,
            disable_bounds_checks=True, disable_semaphore_checks=True),
    )(_sc_body)
    return fn(ph, pt, pr, nh, nt, nr, ent, q)


def _post_body(ss_ref, rel_ref, nv_ref, out_ref):
    s = jnp.sqrt(ss_ref[...])
    basic = jnp.mean(jnp.maximum(_MARGIN + s[:128, :] - s[128:, :], 0.0))
    rel = rel_ref[...]
    nv = nv_ref[...]
    rn = jnp.sqrt(jnp.sum(rel * rel, axis=1))
    wn = jnp.sqrt(jnp.sum(nv * nv, axis=1))
    cons = jnp.sum(jnp.abs(jnp.sum(rel * nv, axis=1) / (rn * wn)))
    out_ref[...] = jnp.broadcast_to(basic + _C_REG * cons, (1, 1))


def _post(ss, rel, nv):
    out = pl.pallas_call(
        _post_body,
        out_shape=jax.ShapeDtypeStruct((1, 1), jnp.float32),
    )(ss, rel, nv)
    return out[0, 0]


def kernel(pos_h, pos_r, pos_t, neg_h, neg_r, neg_t,
           entity_embedding, relation_embedding, normal_vector):
    q = _make_q(relation_embedding, normal_vector)
    ss = _sc_scores(pos_h.astype(jnp.int32), pos_t.astype(jnp.int32),
                    pos_r.astype(jnp.int32), neg_h.astype(jnp.int32),
                    neg_t.astype(jnp.int32), neg_r.astype(jnp.int32),
                    entity_embedding, q)
    return _post(ss, relation_embedding, normal_vector)
